# Initial kernel scaffold; baseline (speedup 1.0000x reference)
import functools

import jax
import jax.numpy as jnp
from jax import lax
from jax.experimental import pallas as pl
from jax.experimental.pallas import tpu as pltpu
from jax.experimental.pallas import tpu_sc as plsc


def _sc_gather(x_idx, tok_table, *, nw, chunks_per_worker, k_inflight):
    d = tok_table.shape[-1]
    n_blocks = nw * chunks_per_worker
    mesh = plsc.VectorSubcoreMesh(core_axis_name="c", subcore_axis_name="s")
    nc = mesh.num_cores

    @functools.partial(
        pl.kernel,
        out_type=jax.ShapeDtypeStruct((n_blocks, 128, d), jnp.float32),
        mesh=mesh,
        scratch_types=[
            pltpu.VMEM((chunks_per_worker, 128), jnp.int32),
            pltpu.VMEM((k_inflight, 128, d), jnp.float32),
            pltpu.SemaphoreType.DMA,
        ],
    )
    def gather_kernel(x_hbm, tok_hbm, out_hbm, idx_v, rows_v, sem):
        wid = lax.axis_index("s") * nc + lax.axis_index("c")
        pltpu.sync_copy(x_hbm.at[wid], idx_v)

        def chunk_body(c, carry):
            g0 = c * k_inflight
            cps = []
            for j in range(k_inflight):
                cp = pltpu.make_async_copy(
                    tok_hbm.at[idx_v.at[g0 + j]], rows_v.at[j], sem
                )
                cp.start()
                cps.append(cp)
            for cp in cps:
                cp.wait()
            pltpu.sync_copy(
                rows_v, out_hbm.at[pl.ds(wid * chunks_per_worker + g0, k_inflight)]
            )
            return carry

        lax.fori_loop(0, chunks_per_worker // k_inflight, chunk_body, 0)

    return gather_kernel(x_idx, tok_table)


def _ln_body(h_ref, pos_ref, g_ref, b_ref, o_ref):
    h = h_ref[...] + pos_ref[...]
    mu = jnp.mean(h, axis=-1, keepdims=True)
    dlt = h - mu
    var = jnp.mean(dlt * dlt, axis=-1, keepdims=True)
    o_ref[...] = dlt * lax.rsqrt(var + 1e-5) * g_ref[...] + b_ref[...]


def _tc_ln(h, pos, gamma, beta, *, bb):
    batch, seq, d = h.shape
    grid = (batch // bb,)
    return pl.pallas_call(
        _ln_body,
        grid=grid,
        in_specs=[
            pl.BlockSpec((bb, seq, d), lambda i: (i, 0, 0)),
            pl.BlockSpec((1, seq, d), lambda i: (0, 0, 0)),
            pl.BlockSpec((1, 1, d), lambda i: (0, 0, 0)),
            pl.BlockSpec((1, 1, d), lambda i: (0, 0, 0)),
        ],
        out_specs=pl.BlockSpec((bb, seq, d), lambda i: (i, 0, 0)),
        out_shape=jax.ShapeDtypeStruct((batch, seq, d), jnp.float32),
    )(h, pos, gamma, beta)


def kernel(x, tok_table, pos_table, gamma, beta):
    batch, seq = x.shape
    d = tok_table.shape[-1]
    n_rows = batch * seq

    nw = 32
    chunks_per_worker = n_rows // (nw * 128)
    x_idx = x.reshape(nw, chunks_per_worker, 128)

    gathered = _sc_gather(
        x_idx, tok_table, nw=nw, chunks_per_worker=chunks_per_worker, k_inflight=4
    )
    h = gathered.reshape(batch, seq, d)

    return _tc_ln(
        h,
        pos_table.reshape(1, seq, d),
        gamma.reshape(1, 1, d),
        beta.reshape(1, 1, d),
        bb=64,
    )


# speedup vs baseline: 2.2631x; 2.2631x over previous
import functools

import jax
import jax.numpy as jnp
from jax import lax
from jax.experimental import pallas as pl
from jax.experimental.pallas import tpu as pltpu
from jax.experimental.pallas import tpu_sc as plsc


def _sc_gather(x_idx, tok_table, *, nw, chunks_per_worker, k_inflight):
    d = tok_table.shape[-1]
    n_blocks = nw * chunks_per_worker
    mesh = plsc.VectorSubcoreMesh(core_axis_name="c", subcore_axis_name="s")
    nc = mesh.num_cores

    @functools.partial(
        pl.kernel,
        out_type=jax.ShapeDtypeStruct((n_blocks, 128, d), jnp.float32),
        mesh=mesh,
        scratch_types=[
            pltpu.VMEM((chunks_per_worker, 128), jnp.int32),
            pltpu.VMEM((k_inflight, 128, d), jnp.float32),
            pltpu.SemaphoreType.DMA,
        ],
        compiler_params=pltpu.CompilerParams(use_tc_tiling_on_sc=False),
    )
    def gather_kernel(x_hbm, tok_hbm, out_hbm, idx_v, rows_v, sem):
        wid = lax.axis_index("s") * nc + lax.axis_index("c")
        pltpu.sync_copy(x_hbm.at[wid], idx_v)

        def chunk_body(c, carry):
            g0 = c * k_inflight
            cps = []
            for j in range(k_inflight):
                cp = pltpu.make_async_copy(
                    tok_hbm.at[idx_v.at[g0 + j]], rows_v.at[j], sem
                )
                cp.start()
                cps.append(cp)
            for cp in cps:
                cp.wait()
            pltpu.sync_copy(
                rows_v, out_hbm.at[pl.ds(wid * chunks_per_worker + g0, k_inflight)]
            )
            return carry

        lax.fori_loop(0, chunks_per_worker // k_inflight, chunk_body, 0)

    return gather_kernel(x_idx, tok_table)


def _ln_body(h_ref, pos_ref, g_ref, b_ref, o_ref):
    h = h_ref[...] + pos_ref[...]
    mu = jnp.mean(h, axis=-1, keepdims=True)
    dlt = h - mu
    var = jnp.mean(dlt * dlt, axis=-1, keepdims=True)
    o_ref[...] = dlt * lax.rsqrt(var + 1e-5) * g_ref[...] + b_ref[...]


def _tc_ln(h, pos, gamma, beta, *, bb):
    batch, seq, d = h.shape
    grid = (batch // bb,)
    return pl.pallas_call(
        _ln_body,
        grid=grid,
        in_specs=[
            pl.BlockSpec((bb, seq, d), lambda i: (i, 0, 0)),
            pl.BlockSpec((1, seq, d), lambda i: (0, 0, 0)),
            pl.BlockSpec((1, 1, d), lambda i: (0, 0, 0)),
            pl.BlockSpec((1, 1, d), lambda i: (0, 0, 0)),
        ],
        out_specs=pl.BlockSpec((bb, seq, d), lambda i: (i, 0, 0)),
        out_shape=jax.ShapeDtypeStruct((batch, seq, d), jnp.float32),
    )(h, pos, gamma, beta)


def kernel(x, tok_table, pos_table, gamma, beta):
    batch, seq = x.shape
    d = tok_table.shape[-1]
    n_rows = batch * seq

    nw = 32
    chunks_per_worker = n_rows // (nw * 128)
    x_idx = x.reshape(nw, chunks_per_worker, 128)

    gathered = _sc_gather(
        x_idx, tok_table, nw=nw, chunks_per_worker=chunks_per_worker, k_inflight=4
    )
    h = gathered.reshape(batch, seq, d)

    return _tc_ln(
        h,
        pos_table.reshape(1, seq, d),
        gamma.reshape(1, 1, d),
        beta.reshape(1, 1, d),
        bb=64,
    )


# speedup vs baseline: 2.2817x; 1.0082x over previous
import functools

import jax
import jax.numpy as jnp
from jax import lax
from jax.experimental import pallas as pl
from jax.experimental.pallas import tpu as pltpu
from jax.experimental.pallas import tpu_sc as plsc


def _sc_gather(x, tok_table, *, nw, rows_per_chunk):
    batch, seq = x.shape
    d = tok_table.shape[-1]
    splits = [(0, 128), (128, seq - 128)]
    bpw = batch // nw
    mesh = plsc.VectorSubcoreMesh(core_axis_name="c", subcore_axis_name="s")
    nc = mesh.num_cores

    @functools.partial(
        pl.kernel,
        out_type=jax.ShapeDtypeStruct((batch, seq, d), jnp.float32),
        mesh=mesh,
        scratch_types=[
            pltpu.VMEM((bpw, seq), jnp.int32),
            pltpu.VMEM((rows_per_chunk, seq, d), jnp.float32),
            pltpu.SemaphoreType.DMA,
        ],
        compiler_params=pltpu.CompilerParams(use_tc_tiling_on_sc=False),
    )
    def gather_kernel(x_hbm, tok_hbm, out_hbm, idx_v, rows_v, sem):
        wid = lax.axis_index("s") * nc + lax.axis_index("c")
        b0 = wid * bpw
        pltpu.sync_copy(x_hbm.at[pl.ds(b0, bpw)], idx_v)

        def chunk_body(i, carry):
            lb = i * rows_per_chunk
            cps = []
            for j in range(rows_per_chunk):
                for off, ln in splits:
                    cp = pltpu.make_async_copy(
                        tok_hbm.at[idx_v.at[lb + j, pl.ds(off, ln)]],
                        rows_v.at[j, pl.ds(off, ln)],
                        sem,
                    )
                    cp.start()
                    cps.append(cp)
            for cp in cps:
                cp.wait()
            pltpu.sync_copy(rows_v, out_hbm.at[pl.ds(b0 + lb, rows_per_chunk)])
            return carry

        lax.fori_loop(0, bpw // rows_per_chunk, chunk_body, 0)

    return gather_kernel(x, tok_table)


def _ln_body(h_ref, pos_ref, g_ref, b_ref, o_ref):
    h = h_ref[...] + pos_ref[...]
    mu = jnp.mean(h, axis=-1, keepdims=True)
    dlt = h - mu
    var = jnp.mean(dlt * dlt, axis=-1, keepdims=True)
    o_ref[...] = dlt * lax.rsqrt(var + 1e-5) * g_ref[...] + b_ref[...]


def _tc_ln(h, pos, gamma, beta, *, bb):
    batch, seq, d = h.shape
    grid = (batch // bb,)
    return pl.pallas_call(
        _ln_body,
        grid=grid,
        in_specs=[
            pl.BlockSpec((bb, seq, d), lambda i: (i, 0, 0)),
            pl.BlockSpec((1, seq, d), lambda i: (0, 0, 0)),
            pl.BlockSpec((1, 1, d), lambda i: (0, 0, 0)),
            pl.BlockSpec((1, 1, d), lambda i: (0, 0, 0)),
        ],
        out_specs=pl.BlockSpec((bb, seq, d), lambda i: (i, 0, 0)),
        out_shape=jax.ShapeDtypeStruct((batch, seq, d), jnp.float32),
    )(h, pos, gamma, beta)


def kernel(x, tok_table, pos_table, gamma, beta):
    batch, seq = x.shape
    d = tok_table.shape[-1]
    n_rows = batch * seq

    nw = 32
    del n_rows

    h = _sc_gather(x, tok_table, nw=nw, rows_per_chunk=4)

    return _tc_ln(
        h,
        pos_table.reshape(1, seq, d),
        gamma.reshape(1, 1, d),
        beta.reshape(1, 1, d),
        bb=64,
    )
